# Initial kernel scaffold; baseline (speedup 1.0000x reference)
import jax
import jax.numpy as jnp
from jax.experimental import pallas as pl


def kernel(x, edge_index, Wl, bl, Wr, br, att, out_bias, gamma, beta):
    raise NotImplementedError("write your pallas kernel here")


# speedup vs baseline: 16.3703x; 16.3703x over previous
import functools

import jax
import jax.numpy as jnp
from jax import lax
from jax.experimental import pallas as pl
from jax.experimental.pallas import tpu as pltpu
from jax.experimental.pallas import tpu_sc as plsc

N = 10000
D = 128
H = 8
C = 16
L = 3
NP = 10240
E0 = 320000
ETOT = E0 + N
NW = 32
K = 64
CHUNKS = 162
PER_W = K * CHUNKS
EP = NW * PER_W
RPT = NP // 16
BM = 512

DR = 1280
RPT2 = 80

_sc_mesh = plsc.VectorSubcoreMesh(core_axis_name="c", subcore_axis_name="s")

_GDN = lax.GatherDimensionNumbers(
    offset_dims=(), collapsed_slice_dims=(0,), start_index_map=(0,))


def _vgather(v, idx):
    return lax.gather(v, idx[:, None], _GDN, (1,),
                      mode=lax.GatherScatterMode.PROMISE_IN_BOUNDS)


@functools.partial(
    pl.kernel,
    mesh=_sc_mesh,
    out_type=[
        jax.ShapeDtypeStruct((2, NP, D), jnp.float32),
        jax.ShapeDtypeStruct((2, DR, D), jnp.float32),
    ],
    scratch_types=[
        pltpu.VMEM((K,), jnp.int32),
        pltpu.VMEM((K,), jnp.int32),
        pltpu.VMEM((K,), jnp.int32),
        pltpu.VMEM((K + 16,), jnp.int32),
        pltpu.VMEM((K, D), jnp.float32),
        pltpu.VMEM((K, D), jnp.float32),
        pltpu.VMEM((K, D), jnp.float32),
        pltpu.VMEM((H, C), jnp.float32),
        pltpu.VMEM_SHARED((NP, D), jnp.float32),
        pltpu.VMEM_SHARED((DR, D), jnp.float32),
        pltpu.SemaphoreType.DMA,
    ],
)
def _edge_kernel(xl_hbm, xr_hbm, src_hbm, dst_hbm, att_hbm, zero_hbm,
                 msg_out, den_out,
                 src_v, dst_v, dst2_v, dstp_v, xlr, xrr, stage2, att_v, accm, accd, sem):
    cid = lax.axis_index("c")
    sid = lax.axis_index("s")
    wid = sid * 2 + cid
    r0 = sid * RPT

    lane = lax.iota(jnp.int32, 16)
    perms = [jnp.bitwise_xor(lane, m) for m in (1, 2, 4, 8)]
    rot8 = jnp.bitwise_and(lane - 8, 15)
    zv = jnp.zeros((16,), jnp.float32)

    def _fill_iota(ref, base0, clamp):
        for m in range(K // 16):
            ref[pl.ds(16 * m, 16)] = jnp.minimum(lane + (base0 + 16 * m), clamp)

    pltpu.sync_copy(zero_hbm, xlr)
    _fill_iota(dst2_v, sid * RPT2, DR - 1)
    pltpu.sync_copy(xlr, accd.at[dst2_v])
    _fill_iota(dst2_v, sid * RPT2 + K, DR - 1)
    pltpu.sync_copy(xlr, accd.at[dst2_v])

    def zero_body(j, c):
        _fill_iota(dst_v, r0 + j * K, NP - 1)
        pltpu.sync_copy(xlr, accm.at[dst_v])
        return c

    lax.fori_loop(0, RPT // K, zero_body, 0)
    pltpu.sync_copy(att_hbm, att_v)
    plsc.subcore_barrier()

    def chunk_body(ci, carry):
        base = pl.multiple_of(wid * PER_W + ci * K, 8)
        pltpu.sync_copy(src_hbm.at[pl.ds(base, K)], src_v)
        pltpu.sync_copy(dst_hbm.at[pl.ds(base, K)], dst_v)
        pltpu.async_copy(xl_hbm.at[src_v], xlr, sem).wait()
        pltpu.async_copy(xr_hbm.at[dst_v], xrr, sem).wait()
        for m in range(K // 16):
            dv = dst_v[pl.ds(16 * m, 16)]
            dst2_v[pl.ds(16 * m, 16)] = lax.shift_right_logical(dv, 3)
            dstp_v[pl.ds(16 * m, 16)] = dv

        def edge_body(i, c2):
            d = zv
            for h in range(H):
                a = xlr[i, pl.ds(h * C, C)]
                b = xrr[i, pl.ds(h * C, C)]
                e = a + b
                e = jnp.maximum(e, 0.2 * e)
                t = e * att_v[h]
                for p in perms:
                    t = t + _vgather(t, p)
                ex = jnp.exp(t)
                xlr[i, pl.ds(h * C, C)] = a * ex
                d = jnp.where(lane == h, ex, d)
            dst_i = dstp_v[pl.ds(i, 16)][0]
            off = jnp.bitwise_and(dst_i, 7) * 16
            for j in range(8):
                stage2[i, pl.ds(16 * j, 16)] = zv
            stage2[i, pl.ds(off, 16)] = d
            return c2

        lax.fori_loop(0, K, edge_body, 0)
        pltpu.sync_copy(xlr, accm.at[dst_v], add=True)
        pltpu.sync_copy(stage2, accd.at[dst2_v], add=True)
        return carry

    lax.fori_loop(0, CHUNKS, chunk_body, 0)
    plsc.subcore_barrier()

    def out_body(j, c):
        _fill_iota(dst_v, r0 + j * K, NP - 1)
        pltpu.async_copy(accm.at[dst_v], xlr, sem).wait()
        pltpu.sync_copy(xlr, msg_out.at[cid, pl.ds(r0 + j * K, K)])
        return c

    lax.fori_loop(0, RPT // K, out_body, 0)
    _fill_iota(dst2_v, sid * RPT2, DR - 1)
    pltpu.async_copy(accd.at[dst2_v], stage2, sem).wait()
    pltpu.sync_copy(stage2, den_out.at[cid, pl.ds(sid * RPT2, K)])
    _fill_iota(dst2_v, sid * RPT2 + K, DR - 1)
    pltpu.async_copy(accd.at[dst2_v], stage2, sem).wait()
    pltpu.sync_copy(stage2.at[pl.ds(0, RPT2 - K)],
                    den_out.at[cid, pl.ds(sid * RPT2 + K, RPT2 - K)])


def _pre_body(h_ref, wl_ref, bl_ref, wr_ref, br_ref, xl_ref, xr_ref):
    hb = h_ref[...]
    xl_ref[...] = jnp.dot(hb, wl_ref[...], preferred_element_type=jnp.float32) + bl_ref[...]
    xr_ref[...] = jnp.dot(hb, wr_ref[...], preferred_element_type=jnp.float32) + br_ref[...]


def _tc_pre(hp, wl, bl, wr, br):
    return pl.pallas_call(
        _pre_body,
        grid=(NP // BM,),
        in_specs=[
            pl.BlockSpec((BM, D), lambda i: (i, 0)),
            pl.BlockSpec((D, D), lambda i: (0, 0)),
            pl.BlockSpec((1, D), lambda i: (0, 0)),
            pl.BlockSpec((D, D), lambda i: (0, 0)),
            pl.BlockSpec((1, D), lambda i: (0, 0)),
        ],
        out_specs=[pl.BlockSpec((BM, D), lambda i: (i, 0))] * 2,
        out_shape=[jax.ShapeDtypeStruct((NP, D), jnp.float32)] * 2,
    )(hp, wl, bl, wr, br)


def _post_body(msg_ref, den_ref, h_ref, ob_ref, gm_ref, bt_ref, out_ref):
    i = pl.program_id(0)
    msg = msg_ref[0] + msg_ref[1]
    den = den_ref[0] + den_ref[1]
    jrow = lax.broadcasted_iota(jnp.int32, (16, D), 0)
    kcol = lax.broadcasted_iota(jnp.int32, (16, D), 1)
    sel = (jrow == kcol // C).astype(jnp.float32)
    denf = jnp.dot(den, sel, preferred_element_type=jnp.float32)
    out = msg / (denf + 1e-16) + ob_ref[...]
    mu = jnp.mean(out, axis=1, keepdims=True)
    var = jnp.mean((out - mu) ** 2, axis=1, keepdims=True)
    y = (out - mu) * lax.rsqrt(var + 1e-5) * gm_ref[...] + bt_ref[...]
    g = jnp.where(y > 0, y, jnp.exp(jnp.minimum(y, 0.0)) - 1.0)
    rows = i * BM + lax.broadcasted_iota(jnp.int32, (BM, 1), 0)
    out_ref[...] = jnp.where(rows < N, h_ref[...] + g, 0.0)


def _tc_post(msgp, den8, hp, ob, gm, bt):
    return pl.pallas_call(
        _post_body,
        grid=(NP // BM,),
        in_specs=[
            pl.BlockSpec((2, BM, D), lambda i: (0, i, 0)),
            pl.BlockSpec((2, BM, 16), lambda i: (0, i, 0)),
            pl.BlockSpec((BM, D), lambda i: (i, 0)),
            pl.BlockSpec((1, D), lambda i: (0, 0)),
            pl.BlockSpec((1, D), lambda i: (0, 0)),
            pl.BlockSpec((1, D), lambda i: (0, 0)),
        ],
        out_specs=pl.BlockSpec((BM, D), lambda i: (i, 0)),
        out_shape=jax.ShapeDtypeStruct((NP, D), jnp.float32),
    )(msgp, den8, hp, ob, gm, bt)


def kernel(x, edge_index, Wl, bl, Wr, br, att, out_bias, gamma, beta):
    f32 = jnp.float32
    hp = jnp.zeros((NP, D), f32).at[:N].set(x.astype(f32))
    loop = jnp.arange(N, dtype=jnp.int32)
    pad = jnp.full((EP - ETOT,), NP - 1, jnp.int32)
    srcp = jnp.concatenate([edge_index[0].astype(jnp.int32), loop, pad])
    dstp = jnp.concatenate([edge_index[1].astype(jnp.int32), loop, pad])
    zero = jnp.zeros((K, D), f32)
    for l in range(L):
        xl, xr = _tc_pre(hp, Wl[l], bl[l].reshape(1, D), Wr[l], br[l].reshape(1, D))
        msgp, denp = _edge_kernel(xl, xr, srcp, dstp, att[l], zero)
        den8 = denp.reshape(2, DR * 8, 16)[:, :NP]
        hp = _tc_post(msgp, den8, hp, out_bias[l].reshape(1, D),
                      gamma[l].reshape(1, D), beta[l].reshape(1, D))
    return hp[:N]


# speedup vs baseline: 20.7425x; 1.2671x over previous
import functools

import jax
import jax.numpy as jnp
from jax import lax
from jax.experimental import pallas as pl
from jax.experimental.pallas import tpu as pltpu
from jax.experimental.pallas import tpu_sc as plsc

N = 10000
D = 128
H = 8
C = 16
L = 3
NP = 10240
E0 = 320000
ETOT = E0 + N
NW = 32
K = 32
CHUNKS = 324
PER_W = K * CHUNKS
EP = NW * PER_W
RPT = NP // 16
DR = 1280
RPT2 = 80
BM = 512

_sc_mesh = plsc.VectorSubcoreMesh(core_axis_name="c", subcore_axis_name="s")

_GDN = lax.GatherDimensionNumbers(
    offset_dims=(), collapsed_slice_dims=(0,), start_index_map=(0,))


def _vgather(v, idx):
    return lax.gather(v, idx[:, None], _GDN, (1,),
                      mode=lax.GatherScatterMode.PROMISE_IN_BOUNDS)


@functools.partial(
    pl.kernel,
    mesh=_sc_mesh,
    out_type=[
        jax.ShapeDtypeStruct((2, NP, D), jnp.float32),
        jax.ShapeDtypeStruct((2, DR, D), jnp.float32),
    ],
    scratch_types=[
        pltpu.VMEM((K,), jnp.int32),
        pltpu.VMEM((K,), jnp.int32),
        pltpu.VMEM((K,), jnp.int32),
        pltpu.VMEM((K,), jnp.int32),
        pltpu.VMEM((K,), jnp.int32),
        pltpu.VMEM((K,), jnp.int32),
        pltpu.VMEM((K,), jnp.int32),
        pltpu.VMEM((K,), jnp.int32),
        pltpu.VMEM((K,), jnp.int32),
        pltpu.VMEM((K,), jnp.int32),
        pltpu.VMEM((K + 16,), jnp.int32),
        pltpu.VMEM((K, D), jnp.float32),
        pltpu.VMEM((K, D), jnp.float32),
        pltpu.VMEM((K, D), jnp.float32),
        pltpu.VMEM((K, D), jnp.float32),
        pltpu.VMEM((K, D), jnp.float32),
        pltpu.VMEM((K, D), jnp.float32),
        pltpu.VMEM((H, C), jnp.float32),
        pltpu.VMEM_SHARED((NP, D), jnp.float32),
        pltpu.VMEM_SHARED((DR, D), jnp.float32),
    ] + [pltpu.SemaphoreType.DMA] * 16,
)
def _edge_kernel(xl_hbm, xr_hbm, src_hbm, dst_hbm, att_hbm, zero_hbm,
                 msg_out, den_out,
                 s0, s1, s2, s3, d0, d1, d2, d3, q0, q1, dstp_v,
                 xlr0, xlr1, xrr0, xrr1, st0, st1, att_v, accm, accd,
                 sgl0, sgl1, sgr0, sgr1, ssm0, ssm1, ssd0, ssd1,
                 si0, si1, si2, si3, sj0, sj1, sj2, sj3):
    src_b = [s0, s1, s2, s3]
    dst_b = [d0, d1, d2, d3]
    q_b = [q0, q1]
    xlr_b = [xlr0, xlr1]
    xrr_b = [xrr0, xrr1]
    st_b = [st0, st1]
    sgl = [sgl0, sgl1]
    sgr = [sgr0, sgr1]
    ssm = [ssm0, ssm1]
    ssd = [ssd0, ssd1]
    sis = [si0, si1, si2, si3]
    sid = [sj0, sj1, sj2, sj3]

    cid = lax.axis_index("c")
    sid_ = lax.axis_index("s")
    wid = sid_ * 2 + cid
    r0 = sid_ * RPT
    r0d = sid_ * RPT2

    lane = lax.iota(jnp.int32, 16)
    perms = [jnp.bitwise_xor(lane, m) for m in (1, 2, 4, 8)]
    zv = jnp.zeros((16,), jnp.float32)

    def _fill_iota(ref, base0, clamp):
        for m in range(K // 16):
            ref[pl.ds(16 * m, 16)] = jnp.minimum(lane + (base0 + 16 * m), clamp)

    pltpu.sync_copy(zero_hbm, xlr0)

    def zero_m(j, c):
        _fill_iota(d0, r0 + j * K, r0 + RPT - 1)
        pltpu.sync_copy(xlr0, accm.at[d0])
        return c

    lax.fori_loop(0, RPT // K, zero_m, 0)

    def zero_d(j, c):
        _fill_iota(d0, r0d + j * K, r0d + RPT2 - 1)
        pltpu.sync_copy(xlr0, accd.at[d0])
        return c

    lax.fori_loop(0, (RPT2 + K - 1) // K, zero_d, 0)
    pltpu.sync_copy(att_hbm, att_v)
    plsc.subcore_barrier()

    emax = EP - K

    def idx_base(i):
        return pl.multiple_of(jnp.minimum(wid * PER_W + i * K, emax), 8)

    def start_idx(i, j):
        b = idx_base(i)
        pltpu.async_copy(src_hbm.at[pl.ds(b, K)], src_b[j], sis[j])
        pltpu.async_copy(dst_hbm.at[pl.ds(b, K)], dst_b[j], sid[j])

    def wait_idx(j):
        pltpu.make_async_copy(src_hbm.at[pl.ds(0, K)], src_b[j], sis[j]).wait()
        pltpu.make_async_copy(dst_hbm.at[pl.ds(0, K)], dst_b[j], sid[j]).wait()

    def start_gather(b, j):
        pltpu.async_copy(xl_hbm.at[src_b[j]], xlr_b[b], sgl[b])
        pltpu.async_copy(xr_hbm.at[dst_b[j]], xrr_b[b], sgr[b])

    def wait_gather(b, j):
        pltpu.make_async_copy(xl_hbm.at[src_b[j]], xlr_b[b], sgl[b]).wait()
        pltpu.make_async_copy(xr_hbm.at[dst_b[j]], xrr_b[b], sgr[b]).wait()

    def wait_msg_scatter(b, j):
        pltpu.make_async_copy(xlr_b[b], accm.at[dst_b[j]], ssm[b]).wait()

    def wait_den_scatter(b):
        pltpu.make_async_copy(st_b[b], accd.at[q_b[b]], ssd[b]).wait()

    def compute(b, j):
        xlr, xrr, st = xlr_b[b], xrr_b[b], st_b[b]
        for m in range(K // 16):
            dv = dst_b[j][pl.ds(16 * m, 16)]
            q_b[b][pl.ds(16 * m, 16)] = lax.shift_right_logical(dv, 3)
            dstp_v[pl.ds(16 * m, 16)] = dv

        def edge_body(i, c2):
            d = zv
            for h in range(H):
                a = xlr[i, pl.ds(h * C, C)]
                bb = xrr[i, pl.ds(h * C, C)]
                e = a + bb
                e = jnp.maximum(e, 0.2 * e)
                t = e * att_v[h]
                for p in perms:
                    t = t + _vgather(t, p)
                ex = jnp.exp(t)
                xlr[i, pl.ds(h * C, C)] = a * ex
                d = jnp.where(lane == h, ex, d)
            dst_i = dstp_v[pl.ds(i, 16)][0]
            off = jnp.bitwise_and(dst_i, 7) * 16
            for jj in range(8):
                st[i, pl.ds(16 * jj, 16)] = zv
            st[i, pl.ds(off, 16)] = d
            return c2

        lax.fori_loop(0, K, edge_body, 0)

    start_idx(0, 0)
    start_idx(1, 1)
    wait_idx(0)
    start_gather(0, 0)

    def quad_body(g, carry):
        for u in range(4):
            i = 4 * g + u
            b = u % 2
            nb = 1 - b
            wait_gather(b, u)
            if u == 0:
                @pl.when(g > 0)
                def _():
                    wait_msg_scatter(nb, 3)
            else:
                wait_msg_scatter(nb, u - 1)
            wait_idx((u + 1) % 4)
            start_gather(nb, (u + 1) % 4)
            if u < 2:
                @pl.when(g > 0)
                def _():
                    wait_den_scatter(b)
            else:
                wait_den_scatter(b)
            start_idx(i + 2, (u + 2) % 4)
            compute(b, u)
            pltpu.async_copy(xlr_b[b], accm.at[dst_b[u]], ssm[b], add=True)
            pltpu.async_copy(st_b[b], accd.at[q_b[b]], ssd[b], add=True)
        return carry

    lax.fori_loop(0, CHUNKS // 4, quad_body, 0)

    wait_msg_scatter(1, 3)
    wait_den_scatter(0)
    wait_den_scatter(1)
    wait_gather(0, 0)
    wait_idx(1)
    plsc.subcore_barrier()

    def out_m(j, c):
        _fill_iota(d0, r0 + j * K, r0 + RPT - 1)
        pltpu.async_copy(accm.at[d0], xlr0, sgl0).wait()
        pltpu.sync_copy(xlr0, msg_out.at[cid, pl.ds(r0 + j * K, K)])
        return c

    lax.fori_loop(0, RPT // K, out_m, 0)

    def out_d(j, c):
        _fill_iota(d0, r0d + j * K, r0d + RPT2 - 1)
        pltpu.async_copy(accd.at[d0], xlr0, sgl0).wait()
        pltpu.sync_copy(xlr0, den_out.at[cid, pl.ds(r0d + j * K, K)])
        return c

    lax.fori_loop(0, RPT2 // K, out_d, 0)
    _fill_iota(d0, r0d + (RPT2 // K) * K, r0d + RPT2 - 1)
    pltpu.async_copy(accd.at[d0], xlr0, sgl0).wait()
    pltpu.sync_copy(xlr0.at[pl.ds(0, RPT2 - (RPT2 // K) * K)],
                    den_out.at[cid, pl.ds(r0d + (RPT2 // K) * K,
                                          RPT2 - (RPT2 // K) * K)])


def _pre_body(h_ref, wl_ref, bl_ref, wr_ref, br_ref, xl_ref, xr_ref):
    hb = h_ref[...]
    xl_ref[...] = jnp.dot(hb, wl_ref[...], preferred_element_type=jnp.float32) + bl_ref[...]
    xr_ref[...] = jnp.dot(hb, wr_ref[...], preferred_element_type=jnp.float32) + br_ref[...]


def _tc_pre(hp, wl, bl, wr, br):
    return pl.pallas_call(
        _pre_body,
        grid=(NP // BM,),
        in_specs=[
            pl.BlockSpec((BM, D), lambda i: (i, 0)),
            pl.BlockSpec((D, D), lambda i: (0, 0)),
            pl.BlockSpec((1, D), lambda i: (0, 0)),
            pl.BlockSpec((D, D), lambda i: (0, 0)),
            pl.BlockSpec((1, D), lambda i: (0, 0)),
        ],
        out_specs=[pl.BlockSpec((BM, D), lambda i: (i, 0))] * 2,
        out_shape=[jax.ShapeDtypeStruct((NP, D), jnp.float32)] * 2,
    )(hp, wl, bl, wr, br)


def _post_body(msg_ref, den_ref, h_ref, ob_ref, gm_ref, bt_ref, out_ref):
    i = pl.program_id(0)
    msg = msg_ref[0] + msg_ref[1]
    den = den_ref[0] + den_ref[1]
    jrow = lax.broadcasted_iota(jnp.int32, (16, D), 0)
    kcol = lax.broadcasted_iota(jnp.int32, (16, D), 1)
    sel = (jrow == kcol // C).astype(jnp.float32)
    denf = jnp.dot(den, sel, preferred_element_type=jnp.float32)
    out = msg / (denf + 1e-16) + ob_ref[...]
    mu = jnp.mean(out, axis=1, keepdims=True)
    var = jnp.mean((out - mu) ** 2, axis=1, keepdims=True)
    y = (out - mu) * lax.rsqrt(var + 1e-5) * gm_ref[...] + bt_ref[...]
    g = jnp.where(y > 0, y, jnp.exp(jnp.minimum(y, 0.0)) - 1.0)
    rows = i * BM + lax.broadcasted_iota(jnp.int32, (BM, 1), 0)
    out_ref[...] = jnp.where(rows < N, h_ref[...] + g, 0.0)


def _tc_post(msgp, den16, hp, ob, gm, bt):
    return pl.pallas_call(
        _post_body,
        grid=(NP // BM,),
        in_specs=[
            pl.BlockSpec((2, BM, D), lambda i: (0, i, 0)),
            pl.BlockSpec((2, BM, 16), lambda i: (0, i, 0)),
            pl.BlockSpec((BM, D), lambda i: (i, 0)),
            pl.BlockSpec((1, D), lambda i: (0, 0)),
            pl.BlockSpec((1, D), lambda i: (0, 0)),
            pl.BlockSpec((1, D), lambda i: (0, 0)),
        ],
        out_specs=pl.BlockSpec((BM, D), lambda i: (i, 0)),
        out_shape=jax.ShapeDtypeStruct((NP, D), jnp.float32),
    )(msgp, den16, hp, ob, gm, bt)


def kernel(x, edge_index, Wl, bl, Wr, br, att, out_bias, gamma, beta):
    f32 = jnp.float32
    hp = jnp.zeros((NP, D), f32).at[:N].set(x.astype(f32))
    loop = jnp.arange(N, dtype=jnp.int32)
    pad = jnp.full((EP - ETOT,), NP - 1, jnp.int32)
    srcp = jnp.concatenate([edge_index[0].astype(jnp.int32), loop, pad])
    dstp = jnp.concatenate([edge_index[1].astype(jnp.int32), loop, pad])
    zero = jnp.zeros((K, D), f32)
    for l in range(L):
        xl, xr = _tc_pre(hp, Wl[l], bl[l].reshape(1, D), Wr[l], br[l].reshape(1, D))
        msgp, denp = _edge_kernel(xl, xr, srcp, dstp, att[l], zero)
        den16 = denp.reshape(2, DR * 8, 16)[:, :NP]
        hp = _tc_post(msgp, den16, hp, out_bias[l].reshape(1, D),
                      gamma[l].reshape(1, D), beta[l].reshape(1, D))
    return hp[:N]


# speedup vs baseline: 55.5791x; 2.6795x over previous
import functools

import jax
import jax.numpy as jnp
from jax import lax
from jax.experimental import pallas as pl
from jax.experimental.pallas import tpu as pltpu
from jax.experimental.pallas import tpu_sc as plsc

N = 10000
D = 128
H = 8
C = 16
L = 3
NP = 10240
E0 = 320000
ETOT = E0 + N
NW = 32
K = 32
CHUNKS = 324
PER_W = K * CHUNKS
EP = NW * PER_W
RPT = NP // 16
DR = 1280
RPT2 = 80
BM = 512

_sc_mesh = plsc.VectorSubcoreMesh(core_axis_name="c", subcore_axis_name="s")

_GDN = lax.GatherDimensionNumbers(
    offset_dims=(), collapsed_slice_dims=(0,), start_index_map=(0,))


def _vgather(v, idx):
    return lax.gather(v, idx[:, None], _GDN, (1,),
                      mode=lax.GatherScatterMode.PROMISE_IN_BOUNDS)


@functools.partial(
    pl.kernel,
    mesh=_sc_mesh,
    out_type=[
        jax.ShapeDtypeStruct((2, NP, D), jnp.float32),
        jax.ShapeDtypeStruct((2, DR, D), jnp.float32),
    ],
    scratch_types=[
        pltpu.VMEM((K,), jnp.int32),
        pltpu.VMEM((K,), jnp.int32),
        pltpu.VMEM((K,), jnp.int32),
        pltpu.VMEM((K,), jnp.int32),
        pltpu.VMEM((K,), jnp.int32),
        pltpu.VMEM((K,), jnp.int32),
        pltpu.VMEM((K,), jnp.int32),
        pltpu.VMEM((K,), jnp.int32),
        pltpu.VMEM((K,), jnp.int32),
        pltpu.VMEM((K,), jnp.int32),
        pltpu.VMEM((K + 16,), jnp.int32),
        pltpu.VMEM((K, D), jnp.float32),
        pltpu.VMEM((K, D), jnp.float32),
        pltpu.VMEM((K, D), jnp.float32),
        pltpu.VMEM((K, D), jnp.float32),
        pltpu.VMEM((K, D), jnp.float32),
        pltpu.VMEM((K, D), jnp.float32),
        pltpu.VMEM((K, D), jnp.float32),
        pltpu.VMEM((K, D), jnp.float32),
        pltpu.VMEM((H, C), jnp.float32),
        pltpu.VMEM_SHARED((NP, D), jnp.float32),
        pltpu.VMEM_SHARED((DR, D), jnp.float32),
    ] + [pltpu.SemaphoreType.DMA] * 16,
)
def _edge_kernel(xl_hbm, xr_hbm, src_hbm, dst_hbm, att_hbm, zero_hbm,
                 msg_out, den_out,
                 s0, s1, s2, s3, d0, d1, d2, d3, q0, q1, dstp_v,
                 xlr0, xlr1, xrr0, xrr1, st0, st1, stg0, stg1, att_v, accm, accd,
                 sgl0, sgl1, sgr0, sgr1, ssm0, ssm1, ssd0, ssd1,
                 si0, si1, si2, si3, sj0, sj1, sj2, sj3):
    src_b = [s0, s1, s2, s3]
    dst_b = [d0, d1, d2, d3]
    q_b = [q0, q1]
    xlr_b = [xlr0, xlr1]
    xrr_b = [xrr0, xrr1]
    st_b = [st0, st1]
    stg_b = [stg0, stg1]
    sgl = [sgl0, sgl1]
    sgr = [sgr0, sgr1]
    ssm = [ssm0, ssm1]
    ssd = [ssd0, ssd1]
    sis = [si0, si1, si2, si3]
    sid = [sj0, sj1, sj2, sj3]

    cid = lax.axis_index("c")
    sid_ = lax.axis_index("s")
    wid = sid_ * 2 + cid
    r0 = sid_ * RPT
    r0d = sid_ * RPT2

    lane = lax.iota(jnp.int32, 16)
    perms = [jnp.bitwise_xor(lane, m) for m in (1, 2, 4, 8)]
    zv = jnp.zeros((16,), jnp.float32)

    def _fill_iota(ref, base0, clamp):
        for m in range(K // 16):
            ref[pl.ds(16 * m, 16)] = jnp.minimum(lane + (base0 + 16 * m), clamp)

    pltpu.sync_copy(zero_hbm, xlr0)

    def zero_m(j, c):
        _fill_iota(d0, r0 + j * K, r0 + RPT - 1)
        pltpu.sync_copy(xlr0, accm.at[d0])
        return c

    lax.fori_loop(0, RPT // K, zero_m, 0)

    def zero_d(j, c):
        _fill_iota(d0, r0d + j * K, r0d + RPT2 - 1)
        pltpu.sync_copy(xlr0, accd.at[d0])
        return c

    lax.fori_loop(0, (RPT2 + K - 1) // K, zero_d, 0)
    pltpu.sync_copy(att_hbm, att_v)
    plsc.subcore_barrier()

    emax = EP - K

    def idx_base(i):
        return pl.multiple_of(jnp.minimum(wid * PER_W + i * K, emax), 8)

    def start_idx(i, j):
        b = idx_base(i)
        pltpu.async_copy(src_hbm.at[pl.ds(b, K)], src_b[j], sis[j])
        pltpu.async_copy(dst_hbm.at[pl.ds(b, K)], dst_b[j], sid[j])

    def wait_idx(j):
        pltpu.make_async_copy(src_hbm.at[pl.ds(0, K)], src_b[j], sis[j]).wait()
        pltpu.make_async_copy(dst_hbm.at[pl.ds(0, K)], dst_b[j], sid[j]).wait()

    def start_gather(b, j):
        pltpu.async_copy(xl_hbm.at[src_b[j]], xlr_b[b], sgl[b])
        pltpu.async_copy(xr_hbm.at[dst_b[j]], xrr_b[b], sgr[b])

    def wait_gather(b, j):
        pltpu.make_async_copy(xl_hbm.at[src_b[j]], xlr_b[b], sgl[b]).wait()
        pltpu.make_async_copy(xr_hbm.at[dst_b[j]], xrr_b[b], sgr[b]).wait()

    def wait_msg_scatter(b, j):
        pltpu.make_async_copy(stg_b[b], accm.at[dst_b[j]], ssm[b]).wait()

    def wait_den_scatter(b):
        pltpu.make_async_copy(st_b[b], accd.at[q_b[b]], ssd[b]).wait()

    def compute(b, j):
        xlr, xrr, st, stg = xlr_b[b], xrr_b[b], st_b[b], stg_b[b]
        for m in range(K // 16):
            dv = dst_b[j][pl.ds(16 * m, 16)]
            q_b[b][pl.ds(16 * m, 16)] = lax.shift_right_logical(dv, 3)
            dstp_v[pl.ds(16 * m, 16)] = dv

        @plsc.parallel_loop(0, K, unroll=2)
        def edge_body(i):
            d = zv
            for h in range(H):
                a = xlr[i, pl.ds(h * C, C)]
                bb = xrr[i, pl.ds(h * C, C)]
                e = a + bb
                e = jnp.maximum(e, 0.2 * e)
                t = e * att_v[h]
                for p in perms:
                    t = t + _vgather(t, p)
                ex = jnp.exp(t)
                stg[i, pl.ds(h * C, C)] = a * ex
                d = jnp.where(lane == h, ex, d)
            dst_i = dstp_v[pl.ds(i, 16)][0]
            off = jnp.bitwise_and(dst_i, 7) * 16
            for jj in range(8):
                st[i, pl.ds(16 * jj, 16)] = zv
            st[i, pl.ds(off, 16)] = d

    start_idx(0, 0)
    start_idx(1, 1)
    wait_idx(0)
    start_gather(0, 0)

    def quad_body(g, carry):
        for u in range(4):
            i = 4 * g + u
            b = u % 2
            nb = 1 - b
            wait_gather(b, u)
            if u == 0:
                @pl.when(g > 0)
                def _():
                    wait_msg_scatter(nb, 3)
            else:
                wait_msg_scatter(nb, u - 1)
            wait_idx((u + 1) % 4)
            start_gather(nb, (u + 1) % 4)
            if u < 2:
                @pl.when(g > 0)
                def _():
                    wait_den_scatter(b)
            else:
                wait_den_scatter(b)
            start_idx(i + 2, (u + 2) % 4)
            compute(b, u)
            pltpu.async_copy(stg_b[b], accm.at[dst_b[u]], ssm[b], add=True)
            pltpu.async_copy(st_b[b], accd.at[q_b[b]], ssd[b], add=True)
        return carry

    lax.fori_loop(0, CHUNKS // 4, quad_body, 0)

    wait_msg_scatter(1, 3)
    wait_den_scatter(0)
    wait_den_scatter(1)
    wait_gather(0, 0)
    wait_idx(1)
    plsc.subcore_barrier()

    def out_m(j, c):
        _fill_iota(d0, r0 + j * K, r0 + RPT - 1)
        pltpu.async_copy(accm.at[d0], xlr0, sgl0).wait()
        pltpu.sync_copy(xlr0, msg_out.at[cid, pl.ds(r0 + j * K, K)])
        return c

    lax.fori_loop(0, RPT // K, out_m, 0)

    def out_d(j, c):
        _fill_iota(d0, r0d + j * K, r0d + RPT2 - 1)
        pltpu.async_copy(accd.at[d0], xlr0, sgl0).wait()
        pltpu.sync_copy(xlr0, den_out.at[cid, pl.ds(r0d + j * K, K)])
        return c

    lax.fori_loop(0, RPT2 // K, out_d, 0)
    _fill_iota(d0, r0d + (RPT2 // K) * K, r0d + RPT2 - 1)
    pltpu.async_copy(accd.at[d0], xlr0, sgl0).wait()
    pltpu.sync_copy(xlr0.at[pl.ds(0, RPT2 - (RPT2 // K) * K)],
                    den_out.at[cid, pl.ds(r0d + (RPT2 // K) * K,
                                          RPT2 - (RPT2 // K) * K)])


def _pre_body(h_ref, wl_ref, bl_ref, wr_ref, br_ref, xl_ref, xr_ref):
    hb = h_ref[...]
    xl_ref[...] = jnp.dot(hb, wl_ref[...], preferred_element_type=jnp.float32) + bl_ref[...]
    xr_ref[...] = jnp.dot(hb, wr_ref[...], preferred_element_type=jnp.float32) + br_ref[...]


def _tc_pre(hp, wl, bl, wr, br):
    return pl.pallas_call(
        _pre_body,
        grid=(NP // BM,),
        in_specs=[
            pl.BlockSpec((BM, D), lambda i: (i, 0)),
            pl.BlockSpec((D, D), lambda i: (0, 0)),
            pl.BlockSpec((1, D), lambda i: (0, 0)),
            pl.BlockSpec((D, D), lambda i: (0, 0)),
            pl.BlockSpec((1, D), lambda i: (0, 0)),
        ],
        out_specs=[pl.BlockSpec((BM, D), lambda i: (i, 0))] * 2,
        out_shape=[jax.ShapeDtypeStruct((NP, D), jnp.float32)] * 2,
    )(hp, wl, bl, wr, br)


def _post_body(msg_ref, den_ref, h_ref, ob_ref, gm_ref, bt_ref, out_ref):
    i = pl.program_id(0)
    msg = msg_ref[0] + msg_ref[1]
    den = den_ref[0] + den_ref[1]
    jrow = lax.broadcasted_iota(jnp.int32, (16, D), 0)
    kcol = lax.broadcasted_iota(jnp.int32, (16, D), 1)
    sel = (jrow == kcol // C).astype(jnp.float32)
    denf = jnp.dot(den, sel, preferred_element_type=jnp.float32)
    out = msg / (denf + 1e-16) + ob_ref[...]
    mu = jnp.mean(out, axis=1, keepdims=True)
    var = jnp.mean((out - mu) ** 2, axis=1, keepdims=True)
    y = (out - mu) * lax.rsqrt(var + 1e-5) * gm_ref[...] + bt_ref[...]
    g = jnp.where(y > 0, y, jnp.exp(jnp.minimum(y, 0.0)) - 1.0)
    rows = i * BM + lax.broadcasted_iota(jnp.int32, (BM, 1), 0)
    out_ref[...] = jnp.where(rows < N, h_ref[...] + g, 0.0)


def _tc_post(msgp, den16, hp, ob, gm, bt):
    return pl.pallas_call(
        _post_body,
        grid=(NP // BM,),
        in_specs=[
            pl.BlockSpec((2, BM, D), lambda i: (0, i, 0)),
            pl.BlockSpec((2, BM, 16), lambda i: (0, i, 0)),
            pl.BlockSpec((BM, D), lambda i: (i, 0)),
            pl.BlockSpec((1, D), lambda i: (0, 0)),
            pl.BlockSpec((1, D), lambda i: (0, 0)),
            pl.BlockSpec((1, D), lambda i: (0, 0)),
        ],
        out_specs=pl.BlockSpec((BM, D), lambda i: (i, 0)),
        out_shape=jax.ShapeDtypeStruct((NP, D), jnp.float32),
    )(msgp, den16, hp, ob, gm, bt)


def kernel(x, edge_index, Wl, bl, Wr, br, att, out_bias, gamma, beta):
    f32 = jnp.float32
    hp = jnp.zeros((NP, D), f32).at[:N].set(x.astype(f32))
    loop = jnp.arange(N, dtype=jnp.int32)
    pad = jnp.full((EP - ETOT,), NP - 1, jnp.int32)
    srcp = jnp.concatenate([edge_index[0].astype(jnp.int32), loop, pad])
    dstp = jnp.concatenate([edge_index[1].astype(jnp.int32), loop, pad])
    zero = jnp.zeros((K, D), f32)
    for l in range(L):
        xl, xr = _tc_pre(hp, Wl[l], bl[l].reshape(1, D), Wr[l], br[l].reshape(1, D))
        msgp, denp = _edge_kernel(xl, xr, srcp, dstp, att[l], zero)
        den16 = denp.reshape(2, DR * 8, 16)[:, :NP]
        hp = _tc_post(msgp, den16, hp, out_bias[l].reshape(1, D),
                      gamma[l].reshape(1, D), beta[l].reshape(1, D))
    return hp[:N]
